# baseline (speedup 1.0000x reference)
import functools

import jax
import jax.numpy as jnp
from jax import lax
from jax.experimental import pallas as pl
from jax.experimental.pallas import tpu as pltpu
from jax.experimental.pallas import tpu_sc as plsc

_K = 16
_TQ = 128
_TN = 256
_D = 80
_CH = 128


def _mm(x, w):
    return lax.dot_general(x, w, (((1,), (1,)), ((), ())),
                           preferred_element_type=jnp.float32)


def _lrelu(x):
    return jnp.where(x >= 0, x, 0.1 * x)


def _relu(x):
    return jnp.maximum(x, 0.0)


def _topk_body(qx_ref, db_ref, idx_ref, *, n_db, k):
    b = pl.program_id(0)
    q = qx_ref[0]
    d = db_ref[0]
    qx, qy, qz = q[:, 0:1], q[:, 1:2], q[:, 2:3]
    dx, dy, dz = d[0:1, :], d[1:2, :], d[2:3, :]
    q2 = qx * qx + qy * qy + qz * qz
    i2 = dx * dx + dy * dy + dz * dz
    rb = lambda v: v.astype(jnp.bfloat16).astype(jnp.float32)
    cross = rb(qx) * rb(dx) + rb(qy) * rb(dy) + rb(qz) * rb(dz)
    dist = (q2 + i2) - 2.0 * cross
    iota = lax.broadcasted_iota(jnp.int32, dist.shape, 1).astype(jnp.float32)
    big = jnp.float32(2 ** 30)
    inf = jnp.float32(jnp.inf)
    cols = []
    for _ in range(k):
        m = jnp.min(dist, axis=1, keepdims=True)
        cand = jnp.where(dist == m, iota, big)
        sel = jnp.min(cand, axis=1, keepdims=True)
        cols.append(sel)
        dist = jnp.where(iota == sel, inf, dist)
    idxf = jnp.concatenate(cols, axis=1)
    idx = idxf.astype(jnp.int32) + b * n_db
    idx_ref[0] = idx


def _knn(queries_t, db, k):
    b_, m_, _ = queries_t.shape
    n_db = db.shape[2]
    tq = _TQ
    return pl.pallas_call(
        functools.partial(_topk_body, n_db=n_db, k=k),
        grid=(b_, m_ // tq),
        in_specs=[
            pl.BlockSpec((1, tq, 3), lambda b, i: (b, i, 0)),
            pl.BlockSpec((1, 3, n_db), lambda b, i: (b, 0, 0)),
        ],
        out_specs=pl.BlockSpec((1, tq, k), lambda b, i: (b, i, 0)),
        out_shape=jax.ShapeDtypeStruct((b_, m_, k), jnp.int32),
    )(queries_t, db)


def _tables_body(f1_ref, f2_ref, x2t_ref, wa_ref, wb_ref, b1_ref,
                 t1_ref, t2_ref):
    f1 = f1_ref[0]
    f2 = f2_ref[0]
    wa = wa_ref[...]
    wb = wb_ref[...]
    t1 = lax.dot_general(f1, wa, (((0,), (1,)), ((), ())),
                         preferred_element_type=jnp.float32)
    t1_ref[0] = t1 + b1_ref[...]
    t2 = lax.dot_general(f2, wb, (((0,), (1,)), ((), ())),
                         preferred_element_type=jnp.float32)
    tn = t2.shape[0]
    pad = jnp.zeros((tn, _D - 67), jnp.float32)
    t2_ref[0] = jnp.concatenate([t2, x2t_ref[0], pad], axis=1)


def _tables(feat1, feat2, xyz2t, wa, wb, b1):
    b_, c_, n_ = feat1.shape
    co = wa.shape[0]
    tn = _TN
    return pl.pallas_call(
        _tables_body,
        grid=(b_, n_ // tn),
        in_specs=[
            pl.BlockSpec((1, c_, tn), lambda b, i: (b, 0, i)),
            pl.BlockSpec((1, c_, tn), lambda b, i: (b, 0, i)),
            pl.BlockSpec((1, tn, 3), lambda b, i: (b, i, 0)),
            pl.BlockSpec((co, c_), lambda b, i: (0, 0)),
            pl.BlockSpec((co, c_), lambda b, i: (0, 0)),
            pl.BlockSpec((1, co), lambda b, i: (0, 0)),
        ],
        out_specs=[
            pl.BlockSpec((1, tn, co), lambda b, i: (b, i, 0)),
            pl.BlockSpec((1, tn, _D), lambda b, i: (b, i, 0)),
        ],
        out_shape=[
            jax.ShapeDtypeStruct((b_, n_, co), jnp.float32),
            jax.ShapeDtypeStruct((b_, n_, _D), jnp.float32),
        ],
    )(feat1, feat2, xyz2t, wa, wb, b1)


def _gather_rows(table, idx):
    v_, d_ = table.shape
    ni = idx.shape[0]
    info = plsc.get_sparse_core_info()
    nw = info.num_cores * info.num_subcores
    per_w = ni // nw
    n_ch = per_w // _CH
    mesh = plsc.VectorSubcoreMesh(core_axis_name="c", subcore_axis_name="s")

    @functools.partial(
        pl.kernel, mesh=mesh,
        out_type=jax.ShapeDtypeStruct((ni, d_), jnp.float32),
        compiler_params=pltpu.CompilerParams(use_tc_tiling_on_sc=False),
        scratch_types=[
            pltpu.VMEM((_CH,), jnp.int32),
            pltpu.VMEM((_CH, d_), jnp.float32),
            pltpu.SemaphoreType.DMA,
        ],
    )
    def k(table_hbm, idx_hbm, out_hbm, idx_v, rows_v, sem):
        wid = lax.axis_index("s") * info.num_cores + lax.axis_index("c")
        base = wid * per_w

        def body(j, carry):
            off = base + j * _CH
            pltpu.sync_copy(idx_hbm.at[pl.ds(off, _CH)], idx_v)
            pltpu.async_copy(table_hbm.at[idx_v], rows_v, sem).wait()
            pltpu.sync_copy(rows_v, out_hbm.at[pl.ds(off, _CH)])
            return carry

        lax.fori_loop(0, n_ch, body, 0)

    return k(table, idx)


def _cost_body(t1_ref, g_ref, x1t_ref, wc_ref, w2_ref, b2_ref,
               wn1w_ref, wn1b_ref, wn2w_ref, wn2b_ref, wn3w_ref, wn3b_ref,
               out_ref, *, k):
    tn = t1_ref.shape[1]
    g = g_ref[...]
    gproj = g[:, 0:64]
    gxyz = g[:, 64:67]
    x1 = x1t_ref[0]
    x1r = jnp.broadcast_to(x1[:, None, :], (tn, k, 3)).reshape(tn * k, 3)
    dxyz = gxyz - x1r
    t1 = t1_ref[0]
    t1r = jnp.broadcast_to(t1[:, None, :], (tn, k, 64)).reshape(tn * k, 64)
    a = _lrelu(gproj + t1r + _mm(dxyz, wc_ref[...]))
    h = _lrelu(_mm(a, w2_ref[...]) + b2_ref[...])
    m = _relu(_mm(dxyz, wn1w_ref[...]) + wn1b_ref[...])
    m = _relu(_mm(m, wn2w_ref[...]) + wn2b_ref[...])
    w = _relu(_mm(m, wn3w_ref[...]) + wn3b_ref[...])
    p2n = (h * w).reshape(tn, k, 64).sum(axis=1)
    pad = jnp.zeros((tn, _D - 67), jnp.float32)
    out_ref[0] = jnp.concatenate([p2n, x1, pad], axis=1)


def _cost_aggr(t1, g12, xyz1t, wc, w2, b2, wn1w, wn1b, wn2w, wn2b, wn3w, wn3b,
               k):
    b_, n_, co = t1.shape
    tn = _TN
    nblk = n_ // tn
    return pl.pallas_call(
        functools.partial(_cost_body, k=k),
        grid=(b_, nblk),
        in_specs=[
            pl.BlockSpec((1, tn, co), lambda b, i: (b, i, 0)),
            pl.BlockSpec((tn * k, _D), lambda b, i, nblk=nblk: (b * nblk + i, 0)),
            pl.BlockSpec((1, tn, 3), lambda b, i: (b, i, 0)),
            pl.BlockSpec(wc.shape, lambda b, i: (0, 0)),
            pl.BlockSpec(w2.shape, lambda b, i: (0, 0)),
            pl.BlockSpec(b2.shape, lambda b, i: (0, 0)),
            pl.BlockSpec(wn1w.shape, lambda b, i: (0, 0)),
            pl.BlockSpec(wn1b.shape, lambda b, i: (0, 0)),
            pl.BlockSpec(wn2w.shape, lambda b, i: (0, 0)),
            pl.BlockSpec(wn2b.shape, lambda b, i: (0, 0)),
            pl.BlockSpec(wn3w.shape, lambda b, i: (0, 0)),
            pl.BlockSpec(wn3b.shape, lambda b, i: (0, 0)),
        ],
        out_specs=pl.BlockSpec((1, tn, _D), lambda b, i: (b, i, 0)),
        out_shape=jax.ShapeDtypeStruct((b_, n_, _D), jnp.float32),
    )(t1, g12, xyz1t, wc, w2, b2, wn1w, wn1b, wn2w, wn2b, wn3w, wn3b)


def _final_body(g_ref, x1t_ref, wn1w_ref, wn1b_ref, wn2w_ref, wn2b_ref,
                wn3w_ref, wn3b_ref, out_ref, *, k):
    tn = x1t_ref.shape[1]
    g = g_ref[...]
    gcost = g[:, 0:64]
    gxyz = g[:, 64:67]
    x1 = x1t_ref[0]
    x1r = jnp.broadcast_to(x1[:, None, :], (tn, k, 3)).reshape(tn * k, 3)
    dxyz = gxyz - x1r
    m = _relu(_mm(dxyz, wn1w_ref[...]) + wn1b_ref[...])
    m = _relu(_mm(m, wn2w_ref[...]) + wn2b_ref[...])
    w = _relu(_mm(m, wn3w_ref[...]) + wn3b_ref[...])
    o = (w * gcost).reshape(tn, k, 64).sum(axis=1)
    out_ref[0] = o.T


def _final(g11, xyz1t, wn1w, wn1b, wn2w, wn2b, wn3w, wn3b, k):
    b_, n_, _ = xyz1t.shape
    co = wn3w.shape[0]
    tn = _TN
    nblk = n_ // tn
    return pl.pallas_call(
        functools.partial(_final_body, k=k),
        grid=(b_, nblk),
        in_specs=[
            pl.BlockSpec((tn * k, _D), lambda b, i, nblk=nblk: (b * nblk + i, 0)),
            pl.BlockSpec((1, tn, 3), lambda b, i: (b, i, 0)),
            pl.BlockSpec(wn1w.shape, lambda b, i: (0, 0)),
            pl.BlockSpec(wn1b.shape, lambda b, i: (0, 0)),
            pl.BlockSpec(wn2w.shape, lambda b, i: (0, 0)),
            pl.BlockSpec(wn2b.shape, lambda b, i: (0, 0)),
            pl.BlockSpec(wn3w.shape, lambda b, i: (0, 0)),
            pl.BlockSpec(wn3b.shape, lambda b, i: (0, 0)),
        ],
        out_specs=pl.BlockSpec((1, co, tn), lambda b, i: (b, 0, i)),
        out_shape=jax.ShapeDtypeStruct((b_, co, n_), jnp.float32),
    )(g11, xyz1t, wn1w, wn1b, wn2w, wn2b, wn3w, wn3b)


def kernel(xyz1, feat1, xyz2, feat2, cost_W1, cost_b1, cost_W2, cost_b2,
           wn1_W1, wn1_b1, wn1_W2, wn1_b2, wn1_W3, wn1_b3,
           wn2_W1, wn2_b1, wn2_W2, wn2_b2, wn2_W3, wn2_b3):
    b_, _, n_ = xyz1.shape
    c_in = feat1.shape[1]
    k = _K
    xyz1t = jnp.transpose(xyz1, (0, 2, 1))
    xyz2t = jnp.transpose(xyz2, (0, 2, 1))

    wa = cost_W1[:, :c_in]
    wb = cost_W1[:, c_in:2 * c_in]
    wc = cost_W1[:, 2 * c_in:]
    b1r = cost_b1.reshape(1, -1)
    b2r = cost_b2.reshape(1, -1)

    idx12 = _knn(xyz1t, xyz2, k)
    t1, t2 = _tables(feat1, feat2, xyz2t, wa, wb, b1r)
    g12 = _gather_rows(t2.reshape(b_ * n_, _D), idx12.reshape(-1))
    idx11 = _knn(xyz1t, xyz1, k)
    p2n = _cost_aggr(t1, g12, xyz1t, wc, cost_W2, b2r,
                     wn2_W1, wn2_b1.reshape(1, -1),
                     wn2_W2, wn2_b2.reshape(1, -1),
                     wn2_W3, wn2_b3.reshape(1, -1), k)

    g11 = _gather_rows(p2n.reshape(b_ * n_, _D), idx11.reshape(-1))
    out = _final(g11, xyz1t,
                 wn1_W1, wn1_b1.reshape(1, -1),
                 wn1_W2, wn1_b2.reshape(1, -1),
                 wn1_W3, wn1_b3.reshape(1, -1), k)
    return out


# speedup vs baseline: 1.1160x; 1.1160x over previous
import functools

import jax
import jax.numpy as jnp
from jax import lax
from jax.experimental import pallas as pl
from jax.experimental.pallas import tpu as pltpu
from jax.experimental.pallas import tpu_sc as plsc

_K = 16
_TQ = 128
_TN = 256
_D = 128
_CH = 128


def _mm(x, w):
    return lax.dot_general(x, w, (((1,), (1,)), ((), ())),
                           preferred_element_type=jnp.float32)


def _lrelu(x):
    return jnp.where(x >= 0, x, 0.1 * x)


def _relu(x):
    return jnp.maximum(x, 0.0)


def _topk_body(qx_ref, db_ref, idx_ref, *, n_db, k):
    b = pl.program_id(0)
    q = qx_ref[0]
    d = db_ref[0]
    qx, qy, qz = q[:, 0:1], q[:, 1:2], q[:, 2:3]
    dx, dy, dz = d[0:1, :], d[1:2, :], d[2:3, :]
    q2 = qx * qx + qy * qy + qz * qz
    i2 = dx * dx + dy * dy + dz * dz
    rb = lambda v: v.astype(jnp.bfloat16).astype(jnp.float32)
    cross = rb(qx) * rb(dx) + rb(qy) * rb(dy) + rb(qz) * rb(dz)
    dist = (q2 + i2) - 2.0 * cross
    iota = lax.broadcasted_iota(jnp.int32, dist.shape, 1).astype(jnp.float32)
    big = jnp.float32(2 ** 30)
    inf = jnp.float32(jnp.inf)
    cols = []
    for _ in range(k):
        m = jnp.min(dist, axis=1, keepdims=True)
        cand = jnp.where(dist == m, iota, big)
        sel = jnp.min(cand, axis=1, keepdims=True)
        cols.append(sel)
        dist = jnp.where(iota == sel, inf, dist)
    idxf = jnp.concatenate(cols, axis=1)
    idx = idxf.astype(jnp.int32) + b * n_db
    idx_ref[0] = idx


def _knn(queries_t, db, k):
    b_, m_, _ = queries_t.shape
    n_db = db.shape[2]
    tq = _TQ
    return pl.pallas_call(
        functools.partial(_topk_body, n_db=n_db, k=k),
        grid=(b_, m_ // tq),
        in_specs=[
            pl.BlockSpec((1, tq, 3), lambda b, i: (b, i, 0)),
            pl.BlockSpec((1, 3, n_db), lambda b, i: (b, 0, 0)),
        ],
        out_specs=pl.BlockSpec((1, tq, k), lambda b, i: (b, i, 0)),
        out_shape=jax.ShapeDtypeStruct((b_, m_, k), jnp.int32),
    )(queries_t, db)


def _tables_body(f1_ref, f2_ref, x2t_ref, wa_ref, wb_ref, b1_ref,
                 t1_ref, t2_ref):
    f1 = f1_ref[0]
    f2 = f2_ref[0]
    wa = wa_ref[...]
    wb = wb_ref[...]
    t1 = lax.dot_general(f1, wa, (((0,), (1,)), ((), ())),
                         preferred_element_type=jnp.float32)
    t1_ref[0] = t1 + b1_ref[...]
    t2 = lax.dot_general(f2, wb, (((0,), (1,)), ((), ())),
                         preferred_element_type=jnp.float32)
    tn = t2.shape[0]
    pad = jnp.zeros((tn, _D - 67), jnp.float32)
    t2_ref[0] = jnp.concatenate([t2, x2t_ref[0], pad], axis=1)


def _tables(feat1, feat2, xyz2t, wa, wb, b1):
    b_, c_, n_ = feat1.shape
    co = wa.shape[0]
    tn = _TN
    return pl.pallas_call(
        _tables_body,
        grid=(b_, n_ // tn),
        in_specs=[
            pl.BlockSpec((1, c_, tn), lambda b, i: (b, 0, i)),
            pl.BlockSpec((1, c_, tn), lambda b, i: (b, 0, i)),
            pl.BlockSpec((1, tn, 3), lambda b, i: (b, i, 0)),
            pl.BlockSpec((co, c_), lambda b, i: (0, 0)),
            pl.BlockSpec((co, c_), lambda b, i: (0, 0)),
            pl.BlockSpec((1, co), lambda b, i: (0, 0)),
        ],
        out_specs=[
            pl.BlockSpec((1, tn, co), lambda b, i: (b, i, 0)),
            pl.BlockSpec((1, tn, _D), lambda b, i: (b, i, 0)),
        ],
        out_shape=[
            jax.ShapeDtypeStruct((b_, n_, co), jnp.float32),
            jax.ShapeDtypeStruct((b_, n_, _D), jnp.float32),
        ],
    )(feat1, feat2, xyz2t, wa, wb, b1)


def _gather_rows(table, idx):
    v_, d_ = table.shape
    ni = idx.shape[0]
    info = plsc.get_sparse_core_info()
    nw = info.num_cores * info.num_subcores
    per_w = ni // nw
    n_ch = per_w // _CH
    mesh = plsc.VectorSubcoreMesh(core_axis_name="c", subcore_axis_name="s")

    @functools.partial(
        pl.kernel, mesh=mesh,
        out_type=jax.ShapeDtypeStruct((ni, d_), jnp.float32),
        scratch_types=[
            pltpu.VMEM((per_w,), jnp.int32),
            pltpu.VMEM((2, _CH, d_), jnp.float32),
            pltpu.SemaphoreType.DMA,
            pltpu.SemaphoreType.DMA,
            pltpu.SemaphoreType.DMA,
        ],
    )
    def k(table_hbm, idx_hbm, out_hbm, idx_all, rows_v, sem_g, sem_o0, sem_o1):
        wid = lax.axis_index("s") * info.num_cores + lax.axis_index("c")
        base = wid * per_w
        pltpu.sync_copy(idx_hbm.at[pl.ds(base, per_w)], idx_all)
        sem_o = (sem_o0, sem_o1)

        def body(j):
            for b in range(2):
                c = j + b
                off = base + c * _CH

                @pl.when(c >= 2)
                def _wait_prev():
                    pltpu.make_async_copy(
                        rows_v.at[b], out_hbm.at[pl.ds(off - 2 * _CH, _CH)],
                        sem_o[b]).wait()

                pltpu.async_copy(
                    table_hbm.at[idx_all.at[pl.ds(c * _CH, _CH)]],
                    rows_v.at[b], sem_g).wait()
                pltpu.async_copy(rows_v.at[b], out_hbm.at[pl.ds(off, _CH)],
                                 sem_o[b])

        pl.loop(0, n_ch, step=2)(body)
        for b in range(2):
            off = base + (n_ch - 2 + b) * _CH
            pltpu.make_async_copy(
                rows_v.at[b], out_hbm.at[pl.ds(off, _CH)], sem_o[b]).wait()

    return k(table, idx)


def _cost_body(t1_ref, g_ref, x1t_ref, wc_ref, w2_ref, b2_ref,
               wn1w_ref, wn1b_ref, wn2w_ref, wn2b_ref, wn3w_ref, wn3b_ref,
               out_ref, *, k):
    tn = t1_ref.shape[1]
    g = g_ref[...]
    gproj = g[:, 0:64]
    gxyz = g[:, 64:67]
    x1 = x1t_ref[0]
    x1r = jnp.broadcast_to(x1[:, None, :], (tn, k, 3)).reshape(tn * k, 3)
    dxyz = gxyz - x1r
    t1 = t1_ref[0]
    t1r = jnp.broadcast_to(t1[:, None, :], (tn, k, 64)).reshape(tn * k, 64)
    a = _lrelu(gproj + t1r + _mm(dxyz, wc_ref[...]))
    h = _lrelu(_mm(a, w2_ref[...]) + b2_ref[...])
    m = _relu(_mm(dxyz, wn1w_ref[...]) + wn1b_ref[...])
    m = _relu(_mm(m, wn2w_ref[...]) + wn2b_ref[...])
    w = _relu(_mm(m, wn3w_ref[...]) + wn3b_ref[...])
    p2n = (h * w).reshape(tn, k, 64).sum(axis=1)
    pad = jnp.zeros((tn, _D - 67), jnp.float32)
    out_ref[0] = jnp.concatenate([p2n, x1, pad], axis=1)


def _cost_aggr(t1, g12, xyz1t, wc, w2, b2, wn1w, wn1b, wn2w, wn2b, wn3w, wn3b,
               k):
    b_, n_, co = t1.shape
    tn = _TN
    nblk = n_ // tn
    return pl.pallas_call(
        functools.partial(_cost_body, k=k),
        grid=(b_, nblk),
        in_specs=[
            pl.BlockSpec((1, tn, co), lambda b, i: (b, i, 0)),
            pl.BlockSpec((tn * k, _D), lambda b, i, nblk=nblk: (b * nblk + i, 0)),
            pl.BlockSpec((1, tn, 3), lambda b, i: (b, i, 0)),
            pl.BlockSpec(wc.shape, lambda b, i: (0, 0)),
            pl.BlockSpec(w2.shape, lambda b, i: (0, 0)),
            pl.BlockSpec(b2.shape, lambda b, i: (0, 0)),
            pl.BlockSpec(wn1w.shape, lambda b, i: (0, 0)),
            pl.BlockSpec(wn1b.shape, lambda b, i: (0, 0)),
            pl.BlockSpec(wn2w.shape, lambda b, i: (0, 0)),
            pl.BlockSpec(wn2b.shape, lambda b, i: (0, 0)),
            pl.BlockSpec(wn3w.shape, lambda b, i: (0, 0)),
            pl.BlockSpec(wn3b.shape, lambda b, i: (0, 0)),
        ],
        out_specs=pl.BlockSpec((1, tn, _D), lambda b, i: (b, i, 0)),
        out_shape=jax.ShapeDtypeStruct((b_, n_, _D), jnp.float32),
    )(t1, g12, xyz1t, wc, w2, b2, wn1w, wn1b, wn2w, wn2b, wn3w, wn3b)


def _final_body(g_ref, x1t_ref, wn1w_ref, wn1b_ref, wn2w_ref, wn2b_ref,
                wn3w_ref, wn3b_ref, out_ref, *, k):
    tn = x1t_ref.shape[1]
    g = g_ref[...]
    gcost = g[:, 0:64]
    gxyz = g[:, 64:67]
    x1 = x1t_ref[0]
    x1r = jnp.broadcast_to(x1[:, None, :], (tn, k, 3)).reshape(tn * k, 3)
    dxyz = gxyz - x1r
    m = _relu(_mm(dxyz, wn1w_ref[...]) + wn1b_ref[...])
    m = _relu(_mm(m, wn2w_ref[...]) + wn2b_ref[...])
    w = _relu(_mm(m, wn3w_ref[...]) + wn3b_ref[...])
    o = (w * gcost).reshape(tn, k, 64).sum(axis=1)
    out_ref[0] = o.T


def _final(g11, xyz1t, wn1w, wn1b, wn2w, wn2b, wn3w, wn3b, k):
    b_, n_, _ = xyz1t.shape
    co = wn3w.shape[0]
    tn = _TN
    nblk = n_ // tn
    return pl.pallas_call(
        functools.partial(_final_body, k=k),
        grid=(b_, nblk),
        in_specs=[
            pl.BlockSpec((tn * k, _D), lambda b, i, nblk=nblk: (b * nblk + i, 0)),
            pl.BlockSpec((1, tn, 3), lambda b, i: (b, i, 0)),
            pl.BlockSpec(wn1w.shape, lambda b, i: (0, 0)),
            pl.BlockSpec(wn1b.shape, lambda b, i: (0, 0)),
            pl.BlockSpec(wn2w.shape, lambda b, i: (0, 0)),
            pl.BlockSpec(wn2b.shape, lambda b, i: (0, 0)),
            pl.BlockSpec(wn3w.shape, lambda b, i: (0, 0)),
            pl.BlockSpec(wn3b.shape, lambda b, i: (0, 0)),
        ],
        out_specs=pl.BlockSpec((1, co, tn), lambda b, i: (b, 0, i)),
        out_shape=jax.ShapeDtypeStruct((b_, co, n_), jnp.float32),
    )(g11, xyz1t, wn1w, wn1b, wn2w, wn2b, wn3w, wn3b)


def kernel(xyz1, feat1, xyz2, feat2, cost_W1, cost_b1, cost_W2, cost_b2,
           wn1_W1, wn1_b1, wn1_W2, wn1_b2, wn1_W3, wn1_b3,
           wn2_W1, wn2_b1, wn2_W2, wn2_b2, wn2_W3, wn2_b3):
    b_, _, n_ = xyz1.shape
    c_in = feat1.shape[1]
    k = _K
    xyz1t = jnp.transpose(xyz1, (0, 2, 1))
    xyz2t = jnp.transpose(xyz2, (0, 2, 1))

    wa = cost_W1[:, :c_in]
    wb = cost_W1[:, c_in:2 * c_in]
    wc = cost_W1[:, 2 * c_in:]
    b1r = cost_b1.reshape(1, -1)
    b2r = cost_b2.reshape(1, -1)

    idx12 = _knn(xyz1t, xyz2, k)
    t1, t2 = _tables(feat1, feat2, xyz2t, wa, wb, b1r)
    g12 = _gather_rows(t2.reshape(b_ * n_, _D), idx12.reshape(-1))
    idx11 = _knn(xyz1t, xyz1, k)
    p2n = _cost_aggr(t1, g12, xyz1t, wc, cost_W2, b2r,
                     wn2_W1, wn2_b1.reshape(1, -1),
                     wn2_W2, wn2_b2.reshape(1, -1),
                     wn2_W3, wn2_b3.reshape(1, -1), k)

    g11 = _gather_rows(p2n.reshape(b_ * n_, _D), idx11.reshape(-1))
    out = _final(g11, xyz1t,
                 wn1_W1, wn1_b1.reshape(1, -1),
                 wn1_W2, wn1_b2.reshape(1, -1),
                 wn1_W3, wn1_b3.reshape(1, -1), k)
    return out


# speedup vs baseline: 1.1518x; 1.0321x over previous
import functools

import jax
import jax.numpy as jnp
from jax import lax
from jax.experimental import pallas as pl
from jax.experimental.pallas import tpu as pltpu
from jax.experimental.pallas import tpu_sc as plsc

_K = 16
_TQ = 256
_TN = 256
_D = 128
_CH = 128


def _mm(x, w):
    return lax.dot_general(x, w, (((1,), (1,)), ((), ())),
                           preferred_element_type=jnp.float32)


def _lrelu(x):
    return jnp.where(x >= 0, x, 0.1 * x)


def _relu(x):
    return jnp.maximum(x, 0.0)


def _topk_body(qx_ref, db_ref, idx_ref, *, n_db, k):
    b = pl.program_id(0)
    q = qx_ref[0]
    d = db_ref[0]
    qx, qy, qz = q[:, 0:1], q[:, 1:2], q[:, 2:3]
    dx, dy, dz = d[0:1, :], d[1:2, :], d[2:3, :]
    q2 = qx * qx + qy * qy + qz * qz
    i2 = dx * dx + dy * dy + dz * dz
    rb = lambda v: v.astype(jnp.bfloat16).astype(jnp.float32)
    cross = rb(qx) * rb(dx) + rb(qy) * rb(dy) + rb(qz) * rb(dz)
    dist = (q2 + i2) - 2.0 * cross
    iota = lax.broadcasted_iota(jnp.int32, dist.shape, 1).astype(jnp.float32)
    big = jnp.float32(2 ** 30)
    inf = jnp.float32(jnp.inf)
    cols = []
    for _ in range(k):
        m = jnp.min(dist, axis=1, keepdims=True)
        cand = jnp.where(dist == m, iota, big)
        sel = jnp.min(cand, axis=1, keepdims=True)
        cols.append(sel)
        dist = jnp.where(iota == sel, inf, dist)
    idxf = jnp.concatenate(cols, axis=1)
    idx = idxf.astype(jnp.int32) + b * n_db
    idx_ref[0] = idx


def _knn(queries_t, db, k):
    b_, m_, _ = queries_t.shape
    n_db = db.shape[2]
    tq = _TQ
    return pl.pallas_call(
        functools.partial(_topk_body, n_db=n_db, k=k),
        grid=(b_, m_ // tq),
        in_specs=[
            pl.BlockSpec((1, tq, 3), lambda b, i: (b, i, 0)),
            pl.BlockSpec((1, 3, n_db), lambda b, i: (b, 0, 0)),
        ],
        out_specs=pl.BlockSpec((1, tq, k), lambda b, i: (b, i, 0)),
        out_shape=jax.ShapeDtypeStruct((b_, m_, k), jnp.int32),
    )(queries_t, db)


def _tables_body(f1_ref, f2_ref, x2t_ref, wa_ref, wb_ref, b1_ref,
                 t1_ref, t2_ref):
    f1 = f1_ref[0]
    f2 = f2_ref[0]
    wa = wa_ref[...]
    wb = wb_ref[...]
    t1 = lax.dot_general(f1, wa, (((0,), (1,)), ((), ())),
                         preferred_element_type=jnp.float32)
    t1_ref[0] = t1 + b1_ref[...]
    t2 = lax.dot_general(f2, wb, (((0,), (1,)), ((), ())),
                         preferred_element_type=jnp.float32)
    tn = t2.shape[0]
    pad = jnp.zeros((tn, _D - 67), jnp.float32)
    t2_ref[0] = jnp.concatenate([t2, x2t_ref[0], pad], axis=1)


def _tables(feat1, feat2, xyz2t, wa, wb, b1):
    b_, c_, n_ = feat1.shape
    co = wa.shape[0]
    tn = _TN
    return pl.pallas_call(
        _tables_body,
        grid=(b_, n_ // tn),
        in_specs=[
            pl.BlockSpec((1, c_, tn), lambda b, i: (b, 0, i)),
            pl.BlockSpec((1, c_, tn), lambda b, i: (b, 0, i)),
            pl.BlockSpec((1, tn, 3), lambda b, i: (b, i, 0)),
            pl.BlockSpec((co, c_), lambda b, i: (0, 0)),
            pl.BlockSpec((co, c_), lambda b, i: (0, 0)),
            pl.BlockSpec((1, co), lambda b, i: (0, 0)),
        ],
        out_specs=[
            pl.BlockSpec((1, tn, co), lambda b, i: (b, i, 0)),
            pl.BlockSpec((1, tn, _D), lambda b, i: (b, i, 0)),
        ],
        out_shape=[
            jax.ShapeDtypeStruct((b_, n_, co), jnp.float32),
            jax.ShapeDtypeStruct((b_, n_, _D), jnp.float32),
        ],
    )(feat1, feat2, xyz2t, wa, wb, b1)


def _gather_rows(table, idx):
    v_, d_ = table.shape
    ni = idx.shape[0]
    info = plsc.get_sparse_core_info()
    nw = info.num_cores * info.num_subcores
    per_w = ni // nw
    n_ch = per_w // _CH
    mesh = plsc.VectorSubcoreMesh(core_axis_name="c", subcore_axis_name="s")

    @functools.partial(
        pl.kernel, mesh=mesh,
        out_type=jax.ShapeDtypeStruct((ni, d_), jnp.float32),
        scratch_types=[
            pltpu.VMEM((per_w,), jnp.int32),
            pltpu.VMEM((2, _CH, d_), jnp.float32),
            pltpu.SemaphoreType.DMA,
            pltpu.SemaphoreType.DMA,
            pltpu.SemaphoreType.DMA,
        ],
    )
    def k(table_hbm, idx_hbm, out_hbm, idx_all, rows_v, sem_g, sem_o0, sem_o1):
        wid = lax.axis_index("s") * info.num_cores + lax.axis_index("c")
        base = wid * per_w
        pltpu.sync_copy(idx_hbm.at[pl.ds(base, per_w)], idx_all)
        sem_o = (sem_o0, sem_o1)

        def body(j):
            for b in range(2):
                c = j + b
                off = base + c * _CH

                @pl.when(c >= 2)
                def _wait_prev():
                    pltpu.make_async_copy(
                        rows_v.at[b], out_hbm.at[pl.ds(off - 2 * _CH, _CH)],
                        sem_o[b]).wait()

                pltpu.async_copy(
                    table_hbm.at[idx_all.at[pl.ds(c * _CH, _CH)]],
                    rows_v.at[b], sem_g).wait()
                pltpu.async_copy(rows_v.at[b], out_hbm.at[pl.ds(off, _CH)],
                                 sem_o[b])

        pl.loop(0, n_ch, step=2)(body)
        for b in range(2):
            off = base + (n_ch - 2 + b) * _CH
            pltpu.make_async_copy(
                rows_v.at[b], out_hbm.at[pl.ds(off, _CH)], sem_o[b]).wait()

    return k(table, idx)


def _cost_body(t1_ref, g_ref, x1t_ref, wc_ref, w2_ref, b2_ref,
               wn1w_ref, wn1b_ref, wn2w_ref, wn2b_ref, wn3w_ref, wn3b_ref,
               out_ref, *, k):
    tn = t1_ref.shape[1]
    g = g_ref[...]
    gproj = g[:, 0:64]
    gxyz = g[:, 64:67]
    x1 = x1t_ref[0]
    x1r = jnp.broadcast_to(x1[:, None, :], (tn, k, 3)).reshape(tn * k, 3)
    dxyz = gxyz - x1r
    t1 = t1_ref[0]
    t1r = jnp.broadcast_to(t1[:, None, :], (tn, k, 64)).reshape(tn * k, 64)
    a = _lrelu(gproj + t1r + _mm(dxyz, wc_ref[...]))
    h = _lrelu(_mm(a, w2_ref[...]) + b2_ref[...])
    m = _relu(_mm(dxyz, wn1w_ref[...]) + wn1b_ref[...])
    m = _relu(_mm(m, wn2w_ref[...]) + wn2b_ref[...])
    w = _relu(_mm(m, wn3w_ref[...]) + wn3b_ref[...])
    p2n = (h * w).reshape(tn, k, 64).sum(axis=1)
    pad = jnp.zeros((tn, _D - 67), jnp.float32)
    out_ref[0] = jnp.concatenate([p2n, x1, pad], axis=1)


def _cost_aggr(t1, g12, xyz1t, wc, w2, b2, wn1w, wn1b, wn2w, wn2b, wn3w, wn3b,
               k):
    b_, n_, co = t1.shape
    tn = _TN
    nblk = n_ // tn
    return pl.pallas_call(
        functools.partial(_cost_body, k=k),
        grid=(b_, nblk),
        in_specs=[
            pl.BlockSpec((1, tn, co), lambda b, i: (b, i, 0)),
            pl.BlockSpec((tn * k, _D), lambda b, i, nblk=nblk: (b * nblk + i, 0)),
            pl.BlockSpec((1, tn, 3), lambda b, i: (b, i, 0)),
            pl.BlockSpec(wc.shape, lambda b, i: (0, 0)),
            pl.BlockSpec(w2.shape, lambda b, i: (0, 0)),
            pl.BlockSpec(b2.shape, lambda b, i: (0, 0)),
            pl.BlockSpec(wn1w.shape, lambda b, i: (0, 0)),
            pl.BlockSpec(wn1b.shape, lambda b, i: (0, 0)),
            pl.BlockSpec(wn2w.shape, lambda b, i: (0, 0)),
            pl.BlockSpec(wn2b.shape, lambda b, i: (0, 0)),
            pl.BlockSpec(wn3w.shape, lambda b, i: (0, 0)),
            pl.BlockSpec(wn3b.shape, lambda b, i: (0, 0)),
        ],
        out_specs=pl.BlockSpec((1, tn, _D), lambda b, i: (b, i, 0)),
        out_shape=jax.ShapeDtypeStruct((b_, n_, _D), jnp.float32),
    )(t1, g12, xyz1t, wc, w2, b2, wn1w, wn1b, wn2w, wn2b, wn3w, wn3b)


def _final_body(g_ref, x1t_ref, wn1w_ref, wn1b_ref, wn2w_ref, wn2b_ref,
                wn3w_ref, wn3b_ref, out_ref, *, k):
    tn = x1t_ref.shape[1]
    g = g_ref[...]
    gcost = g[:, 0:64]
    gxyz = g[:, 64:67]
    x1 = x1t_ref[0]
    x1r = jnp.broadcast_to(x1[:, None, :], (tn, k, 3)).reshape(tn * k, 3)
    dxyz = gxyz - x1r
    m = _relu(_mm(dxyz, wn1w_ref[...]) + wn1b_ref[...])
    m = _relu(_mm(m, wn2w_ref[...]) + wn2b_ref[...])
    w = _relu(_mm(m, wn3w_ref[...]) + wn3b_ref[...])
    o = (w * gcost).reshape(tn, k, 64).sum(axis=1)
    out_ref[0] = o.T


def _final(g11, xyz1t, wn1w, wn1b, wn2w, wn2b, wn3w, wn3b, k):
    b_, n_, _ = xyz1t.shape
    co = wn3w.shape[0]
    tn = _TN
    nblk = n_ // tn
    return pl.pallas_call(
        functools.partial(_final_body, k=k),
        grid=(b_, nblk),
        in_specs=[
            pl.BlockSpec((tn * k, _D), lambda b, i, nblk=nblk: (b * nblk + i, 0)),
            pl.BlockSpec((1, tn, 3), lambda b, i: (b, i, 0)),
            pl.BlockSpec(wn1w.shape, lambda b, i: (0, 0)),
            pl.BlockSpec(wn1b.shape, lambda b, i: (0, 0)),
            pl.BlockSpec(wn2w.shape, lambda b, i: (0, 0)),
            pl.BlockSpec(wn2b.shape, lambda b, i: (0, 0)),
            pl.BlockSpec(wn3w.shape, lambda b, i: (0, 0)),
            pl.BlockSpec(wn3b.shape, lambda b, i: (0, 0)),
        ],
        out_specs=pl.BlockSpec((1, co, tn), lambda b, i: (b, 0, i)),
        out_shape=jax.ShapeDtypeStruct((b_, co, n_), jnp.float32),
    )(g11, xyz1t, wn1w, wn1b, wn2w, wn2b, wn3w, wn3b)


def kernel(xyz1, feat1, xyz2, feat2, cost_W1, cost_b1, cost_W2, cost_b2,
           wn1_W1, wn1_b1, wn1_W2, wn1_b2, wn1_W3, wn1_b3,
           wn2_W1, wn2_b1, wn2_W2, wn2_b2, wn2_W3, wn2_b3):
    b_, _, n_ = xyz1.shape
    c_in = feat1.shape[1]
    k = _K
    xyz1t = jnp.transpose(xyz1, (0, 2, 1))
    xyz2t = jnp.transpose(xyz2, (0, 2, 1))

    wa = cost_W1[:, :c_in]
    wb = cost_W1[:, c_in:2 * c_in]
    wc = cost_W1[:, 2 * c_in:]
    b1r = cost_b1.reshape(1, -1)
    b2r = cost_b2.reshape(1, -1)

    idx12 = _knn(xyz1t, xyz2, k)
    t1, t2 = _tables(feat1, feat2, xyz2t, wa, wb, b1r)
    g12 = _gather_rows(t2.reshape(b_ * n_, _D), idx12.reshape(-1))
    idx11 = _knn(xyz1t, xyz1, k)
    p2n = _cost_aggr(t1, g12, xyz1t, wc, cost_W2, b2r,
                     wn2_W1, wn2_b1.reshape(1, -1),
                     wn2_W2, wn2_b2.reshape(1, -1),
                     wn2_W3, wn2_b3.reshape(1, -1), k)

    g11 = _gather_rows(p2n.reshape(b_ * n_, _D), idx11.reshape(-1))
    out = _final(g11, xyz1t,
                 wn1_W1, wn1_b1.reshape(1, -1),
                 wn1_W2, wn1_b2.reshape(1, -1),
                 wn1_W3, wn1_b3.reshape(1, -1), k)
    return out


# speedup vs baseline: 1.2248x; 1.0634x over previous
import functools

import jax
import jax.numpy as jnp
from jax import lax
from jax.experimental import pallas as pl
from jax.experimental.pallas import tpu as pltpu
from jax.experimental.pallas import tpu_sc as plsc

_K = 16
_TQ = 256
_TN = 256
_D = 128
_DU = 72
_CH = 128


def _mm(x, w):
    return lax.dot_general(x, w, (((1,), (1,)), ((), ())),
                           preferred_element_type=jnp.float32)


def _mtm(f, w):
    return lax.dot_general(f, w, (((0,), (1,)), ((), ())),
                           preferred_element_type=jnp.float32)


def _lrelu(x):
    return jnp.where(x >= 0, x, 0.1 * x)


def _relu(x):
    return jnp.maximum(x, 0.0)


def _bcast_k(x, k):
    r, w = x.shape
    return jnp.broadcast_to(x[:, None, :], (r, k, w)).reshape(r * k, w)


def _topk(qx, db, k, base):
    qxc, qyc, qzc = qx[:, 0:1], qx[:, 1:2], qx[:, 2:3]
    dx, dy, dz = db[0:1, :], db[1:2, :], db[2:3, :]
    q2 = qxc * qxc + qyc * qyc + qzc * qzc
    i2 = dx * dx + dy * dy + dz * dz
    rb = lambda v: v.astype(jnp.bfloat16).astype(jnp.float32)
    cross = rb(qxc) * rb(dx) + rb(qyc) * rb(dy) + rb(qzc) * rb(dz)
    dist = (q2 + i2) - 2.0 * cross
    iota = lax.broadcasted_iota(jnp.int32, dist.shape, 1).astype(jnp.float32)
    big = jnp.float32(2 ** 30)
    inf = jnp.float32(jnp.inf)
    cols = []
    for _ in range(k):
        m = jnp.min(dist, axis=1, keepdims=True)
        cand = jnp.where(dist == m, iota, big)
        sel = jnp.min(cand, axis=1, keepdims=True)
        cols.append(sel)
        dist = jnp.where(iota == sel, inf, dist)
    return jnp.concatenate(cols, axis=1).astype(jnp.int32) + base


def _knn12_tables_body(x1_ref, db_ref, f1_ref, f2_ref, wa_ref, wb_ref,
                       wc_ref, b1_ref, wn2w1_ref, wn2b1_ref,
                       idx_ref, t1_ref, t2_ref, *, n_db, k, tq):
    b = pl.program_id(0)
    j = pl.program_id(1)
    x1t = jnp.transpose(x1_ref[0], (1, 0))
    db = db_ref[0]
    idx_ref[0] = _topk(x1t, db, k, b * n_db)

    wc = wc_ref[...]
    wn2w1 = wn2w1_ref[...]
    t1 = _mtm(f1_ref[0], wa_ref[...]) + b1_ref[...] - _mm(x1t, wc)
    u2 = wn2b1_ref[...] - _mm(x1t, wn2w1)
    t1_ref[0] = jnp.concatenate([t1, u2], axis=1)
    x2t = jnp.transpose(db_ref[0, :, pl.ds(j * tq, tq)], (1, 0))
    t2 = _mtm(f2_ref[0], wb_ref[...]) + _mm(x2t, wc)
    w2p = _mm(x2t, wn2w1)
    t2_ref[0, :, 0:_DU] = jnp.concatenate([t2, w2p], axis=1)


def _knn12_tables(xyz1, xyz2, feat1, feat2, wa, wb, wc, b1, wn2w1, wn2b1, k):
    b_, c_, n_ = feat1.shape
    tq = _TQ
    return pl.pallas_call(
        functools.partial(_knn12_tables_body, n_db=n_, k=k, tq=tq),
        grid=(b_, n_ // tq),
        in_specs=[
            pl.BlockSpec((1, 3, tq), lambda b, i: (b, 0, i)),
            pl.BlockSpec((1, 3, n_), lambda b, i: (b, 0, 0)),
            pl.BlockSpec((1, c_, tq), lambda b, i: (b, 0, i)),
            pl.BlockSpec((1, c_, tq), lambda b, i: (b, 0, i)),
            pl.BlockSpec(wa.shape, lambda b, i: (0, 0)),
            pl.BlockSpec(wb.shape, lambda b, i: (0, 0)),
            pl.BlockSpec(wc.shape, lambda b, i: (0, 0)),
            pl.BlockSpec(b1.shape, lambda b, i: (0, 0)),
            pl.BlockSpec(wn2w1.shape, lambda b, i: (0, 0)),
            pl.BlockSpec(wn2b1.shape, lambda b, i: (0, 0)),
        ],
        out_specs=[
            pl.BlockSpec((1, tq, k), lambda b, i: (b, i, 0)),
            pl.BlockSpec((1, tq, _DU), lambda b, i: (b, i, 0)),
            pl.BlockSpec((1, tq, _D), lambda b, i: (b, i, 0)),
        ],
        out_shape=[
            jax.ShapeDtypeStruct((b_, n_, k), jnp.int32),
            jax.ShapeDtypeStruct((b_, n_, _DU), jnp.float32),
            jax.ShapeDtypeStruct((b_, n_, _D), jnp.float32),
        ],
    )(xyz1, xyz2, feat1, feat2, wa, wb, wc, b1, wn2w1, wn2b1)


def _knn_body(x1_ref, db_ref, idx_ref, *, n_db, k):
    b = pl.program_id(0)
    x1t = jnp.transpose(x1_ref[0], (1, 0))
    idx_ref[0] = _topk(x1t, db_ref[0], k, b * n_db)


def _knn(xyz1, k):
    b_, _, n_ = xyz1.shape
    tq = _TQ
    return pl.pallas_call(
        functools.partial(_knn_body, n_db=n_, k=k),
        grid=(b_, n_ // tq),
        in_specs=[
            pl.BlockSpec((1, 3, tq), lambda b, i: (b, 0, i)),
            pl.BlockSpec((1, 3, n_), lambda b, i: (b, 0, 0)),
        ],
        out_specs=pl.BlockSpec((1, tq, k), lambda b, i: (b, i, 0)),
        out_shape=jax.ShapeDtypeStruct((b_, n_, k), jnp.int32),
    )(xyz1, xyz1)


def _gather_rows(table, idx):
    v_, d_ = table.shape
    ni = idx.shape[0]
    info = plsc.get_sparse_core_info()
    nw = info.num_cores * info.num_subcores
    per_w = ni // nw
    n_ch = per_w // _CH
    mesh = plsc.VectorSubcoreMesh(core_axis_name="c", subcore_axis_name="s")

    @functools.partial(
        pl.kernel, mesh=mesh,
        out_type=jax.ShapeDtypeStruct((ni, d_), jnp.float32),
        scratch_types=[
            pltpu.VMEM((per_w,), jnp.int32),
            pltpu.VMEM((2, _CH, d_), jnp.float32),
            pltpu.SemaphoreType.DMA,
            pltpu.SemaphoreType.DMA,
            pltpu.SemaphoreType.DMA,
        ],
    )
    def k(table_hbm, idx_hbm, out_hbm, idx_all, rows_v, sem_g, sem_o0, sem_o1):
        wid = lax.axis_index("s") * info.num_cores + lax.axis_index("c")
        base = wid * per_w
        pltpu.sync_copy(idx_hbm.at[pl.ds(base, per_w)], idx_all)
        sem_o = (sem_o0, sem_o1)

        def body(j):
            for b in range(2):
                c = j + b
                off = base + c * _CH

                @pl.when(c >= 2)
                def _wait_prev():
                    pltpu.make_async_copy(
                        rows_v.at[b], out_hbm.at[pl.ds(off - 2 * _CH, _CH)],
                        sem_o[b]).wait()

                pltpu.async_copy(
                    table_hbm.at[idx_all.at[pl.ds(c * _CH, _CH)]],
                    rows_v.at[b], sem_g).wait()
                pltpu.async_copy(rows_v.at[b], out_hbm.at[pl.ds(off, _CH)],
                                 sem_o[b])

        pl.loop(0, n_ch, step=2)(body)
        for b in range(2):
            off = base + (n_ch - 2 + b) * _CH
            pltpu.make_async_copy(
                rows_v.at[b], out_hbm.at[pl.ds(off, _CH)], sem_o[b]).wait()

    return k(table, idx)


def _cost_body(t1_ref, g_ref, x1_ref, w2_ref, b2_ref,
               wn2w2_ref, wn2b2_ref, wn2w3_ref, wn2b3_ref, wn1w1_ref,
               out_ref, *, k):
    tn = t1_ref.shape[1]
    g = g_ref[...]
    base = _bcast_k(t1_ref[0], k)
    a = _lrelu(g[:, 0:64] + base[:, 0:64])
    h = _lrelu(_mm(a, w2_ref[...]) + b2_ref[...])
    m = _relu(g[:, 64:_DU] + base[:, 64:_DU])
    m = _relu(_mm(m, wn2w2_ref[...]) + wn2b2_ref[...])
    w = _relu(_mm(m, wn2w3_ref[...]) + wn2b3_ref[...])
    p2n = (h * w).reshape(tn, k, 64).sum(axis=1)
    x1t = jnp.transpose(x1_ref[0], (1, 0))
    w1p = _mm(x1t, wn1w1_ref[...])
    out_ref[0, :, 0:_DU] = jnp.concatenate([p2n, w1p], axis=1)


def _cost_aggr(t1ext, g12, xyz1, w2, b2, wn2w2, wn2b2, wn2w3, wn2b3, wn1w1,
               k):
    b_, n_, _ = t1ext.shape
    tn = _TN
    nblk = n_ // tn
    return pl.pallas_call(
        functools.partial(_cost_body, k=k),
        grid=(b_, nblk),
        in_specs=[
            pl.BlockSpec((1, tn, _DU), lambda b, i: (b, i, 0)),
            pl.BlockSpec((tn * k, _D), lambda b, i, nblk=nblk: (b * nblk + i, 0)),
            pl.BlockSpec((1, 3, tn), lambda b, i: (b, 0, i)),
            pl.BlockSpec(w2.shape, lambda b, i: (0, 0)),
            pl.BlockSpec(b2.shape, lambda b, i: (0, 0)),
            pl.BlockSpec(wn2w2.shape, lambda b, i: (0, 0)),
            pl.BlockSpec(wn2b2.shape, lambda b, i: (0, 0)),
            pl.BlockSpec(wn2w3.shape, lambda b, i: (0, 0)),
            pl.BlockSpec(wn2b3.shape, lambda b, i: (0, 0)),
            pl.BlockSpec(wn1w1.shape, lambda b, i: (0, 0)),
        ],
        out_specs=pl.BlockSpec((1, tn, _D), lambda b, i: (b, i, 0)),
        out_shape=jax.ShapeDtypeStruct((b_, n_, _D), jnp.float32),
    )(t1ext, g12, xyz1, w2, b2, wn2w2, wn2b2, wn2w3, wn2b3, wn1w1)


def _final_body(g_ref, q_ref, wn1b1_ref, wn1w2_ref, wn1b2_ref,
                wn1w3_ref, wn1b3_ref, out_ref, *, k):
    tn = q_ref.shape[1]
    g = g_ref[...]
    u1 = wn1b1_ref[...] - q_ref[0, :, 64:_DU]
    m = _relu(g[:, 64:_DU] + _bcast_k(u1, k))
    m = _relu(_mm(m, wn1w2_ref[...]) + wn1b2_ref[...])
    w = _relu(_mm(m, wn1w3_ref[...]) + wn1b3_ref[...])
    o = (w * g[:, 0:64]).reshape(tn, k, 64).sum(axis=1)
    out_ref[0] = jnp.transpose(o, (1, 0))


def _final_b(g11, p2ntab_b, wn1b1, wn1w2, wn1b2, wn1w3, wn1b3, k):
    n_ = p2ntab_b.shape[1]
    co = wn1w3.shape[0]
    tn = _TN
    return pl.pallas_call(
        functools.partial(_final_body, k=k),
        grid=(n_ // tn,),
        in_specs=[
            pl.BlockSpec((tn * k, _D), lambda i: (i, 0)),
            pl.BlockSpec((1, tn, _D), lambda i: (0, i, 0)),
            pl.BlockSpec(wn1b1.shape, lambda i: (0, 0)),
            pl.BlockSpec(wn1w2.shape, lambda i: (0, 0)),
            pl.BlockSpec(wn1b2.shape, lambda i: (0, 0)),
            pl.BlockSpec(wn1w3.shape, lambda i: (0, 0)),
            pl.BlockSpec(wn1b3.shape, lambda i: (0, 0)),
        ],
        out_specs=pl.BlockSpec((1, co, tn), lambda i: (0, 0, i)),
        out_shape=jax.ShapeDtypeStruct((1, co, n_), jnp.float32),
    )(g11, p2ntab_b, wn1b1, wn1w2, wn1b2, wn1w3, wn1b3)


def kernel(xyz1, feat1, xyz2, feat2, cost_W1, cost_b1, cost_W2, cost_b2,
           wn1_W1, wn1_b1, wn1_W2, wn1_b2, wn1_W3, wn1_b3,
           wn2_W1, wn2_b1, wn2_W2, wn2_b2, wn2_W3, wn2_b3):
    b_, _, n_ = xyz1.shape
    c_in = feat1.shape[1]
    k = _K

    wa = cost_W1[:, :c_in]
    wb = cost_W1[:, c_in:2 * c_in]
    wc = cost_W1[:, 2 * c_in:]
    r1 = lambda v: v.reshape(1, -1)

    idx12, t1ext, t2tab = _knn12_tables(
        xyz1, xyz2, feat1, feat2, wa, wb, wc, r1(cost_b1),
        wn2_W1, r1(wn2_b1), k)
    g12 = _gather_rows(t2tab.reshape(b_ * n_, _D), idx12.reshape(-1))
    idx11 = _knn(xyz1, k)

    p2ntab = _cost_aggr(t1ext, g12, xyz1, cost_W2, r1(cost_b2),
                        wn2_W2, r1(wn2_b2), wn2_W3, r1(wn2_b3), wn1_W1, k)

    p2nflat = p2ntab.reshape(b_ * n_, _D)
    outs = []
    for b in range(b_):
        g11 = _gather_rows(p2nflat, idx11[b].reshape(-1))
        outs.append(_final_b(g11, p2ntab[b:b + 1], r1(wn1_b1),
                             wn1_W2, r1(wn1_b2), wn1_W3, r1(wn1_b3), k))
    return jnp.concatenate(outs, axis=0)
